# baseline (speedup 1.0000x reference)
import functools

import jax
import jax.numpy as jnp
from jax import lax
from jax.experimental import pallas as pl
from jax.experimental.pallas import tpu as pltpu
from jax.experimental.pallas import tpu_sc as plsc

_N = 10000
_E = 320000
_NC, _NS, _L = 2, 16, 16
_NW = _NC * _NS
_B = 128
_KB = 8
_TB = 80
_E_PAD = _NW * _TB * _B
_TB0_AGG, _TB1_AGG = 64, 96
_TB0_DEG, _TB1_DEG = 80, 80
_NPAD = 10016
_RPS = _N // _NS
_RB = 5000
_G = _N // _RB

_mesh = plsc.VectorSubcoreMesh(core_axis_name="c", subcore_axis_name="s",
                               num_cores=_NC, num_subcores=_NS)
_sc_params = pltpu.CompilerParams(needs_layout_passes=False,
                                  use_tc_tiling_on_sc=False,
                                  skip_device_barrier=True)


@functools.partial(
    pl.kernel,
    out_type=jax.ShapeDtypeStruct((_NW, _N), jnp.float32),
    mesh=_mesh,
    scratch_types=[
        pltpu.VMEM((_NPAD,), jnp.float32),
        pltpu.VMEM((_KB, _B), jnp.int32),
        pltpu.VMEM((_KB, _B), jnp.int32),
        pltpu.SemaphoreType.DMA,
        pltpu.SemaphoreType.DMA,
    ],
    compiler_params=_sc_params,
)
def _deg_sc(dst2d, out, hist, buf0, buf1, isem0, isem1):
    bufs, isems = [buf0, buf1], [isem0, isem1]
    c = lax.axis_index("c")
    s = lax.axis_index("s")
    wid = c * _NS + s
    base = lax.select(c == 0, s * _TB0_DEG, _NS * _TB0_DEG + s * _TB1_DEG)
    nch = lax.select(c == 0, _TB0_DEG // _KB, _TB1_DEG // _KB)
    zeros16 = jnp.zeros((_L,), jnp.float32)
    ones16 = jnp.ones((_L,), jnp.float32)

    for b in range(2):
        pltpu.async_copy(dst2d.at[pl.ds(base + b * _KB, _KB)], bufs[b],
                         isems[b])

    @pl.loop(0, _NPAD // _L)
    def _zero(i):
        hist[pl.ds(i * _L, _L)] = zeros16

    @pl.loop(0, nch // 2)
    def _chunk(i):
        ch0 = i * 2
        for b in range(2):
            pltpu.make_async_copy(dst2d.at[pl.ds(0, _KB)], bufs[b],
                                  isems[b]).wait()
            for j in range(_KB):
                for v in range(_B // _L):
                    idx = bufs[b][j, pl.ds(v * _L, _L)]
                    plsc.addupdate_scatter(hist, [idx], ones16)

            @pl.when(ch0 + b + 2 < nch)
            def _pref():
                row = base + (ch0 + b + 2) * _KB
                pltpu.async_copy(dst2d.at[pl.ds(row, _KB)], bufs[b], isems[b])

    pltpu.sync_copy(hist.at[pl.ds(0, _N)], out.at[wid])


def _make_agg(d, dtype):
    @functools.partial(
        pl.kernel,
        out_type=jax.ShapeDtypeStruct((_NC, _N, d), dtype),
        mesh=_mesh,
        scratch_types=[
            pltpu.VMEM((_KB, _B), jnp.int32),
            pltpu.VMEM((_KB, _B), jnp.int32),
            pltpu.VMEM((_KB, _B), jnp.int32),
            pltpu.VMEM((_KB, _B), jnp.int32),
            pltpu.VMEM((_KB, _B, d), dtype),
            pltpu.VMEM((_KB, _B, d), dtype),
            pltpu.VMEM_SHARED((_NPAD, d), dtype),
            pltpu.VMEM_SHARED((_N, d), dtype),
            pltpu.SemaphoreType.DMA,
            pltpu.SemaphoreType.DMA,
            pltpu.SemaphoreType.DMA,
            pltpu.SemaphoreType.DMA,
            pltpu.SemaphoreType.DMA,
            pltpu.SemaphoreType.DMA,
        ],
        compiler_params=_sc_params,
    )
    def _agg(table, src2d, dst2d, zeros, out, s0, s1, d0, d1, r0, r1, acc,
             tab_s, is0, is1, gs0, gs1, ss0, ss1):
        sb, db, rb = [s0, s1], [d0, d1], [r0, r1]
        isem, gsem, ssem = [is0, is1], [gs0, gs1], [ss0, ss1]
        c = lax.axis_index("c")
        s = lax.axis_index("s")
        base = lax.select(c == 0, s * _TB0_AGG,
                          _NS * _TB0_AGG + s * _TB1_AGG)
        nch = lax.select(c == 0, _TB0_AGG // _KB, _TB1_AGG // _KB)

        for b in range(2):
            row = base + b * _KB
            pltpu.async_copy(src2d.at[pl.ds(row, _KB)], sb[b], isem[b])
            pltpu.async_copy(dst2d.at[pl.ds(row, _KB)], db[b], isem[b])
        pltpu.sync_copy(zeros.at[pl.ds(s * _RPS, _RPS)],
                        acc.at[pl.ds(s * _RPS, _RPS)])
        pltpu.sync_copy(table.at[pl.ds(s * _RPS, _RPS)],
                        tab_s.at[pl.ds(s * _RPS, _RPS)])
        plsc.subcore_barrier()

        @pl.loop(0, nch // 2)
        def _chunk(i):
            ch0 = i * 2
            gets = [None, None]
            puts = [None, None]
            for b in range(2):
                pltpu.make_async_copy(src2d.at[pl.ds(0, _KB)], sb[b],
                                      isem[b]).wait()
                pltpu.make_async_copy(dst2d.at[pl.ds(0, _KB)], db[b],
                                      isem[b]).wait()
                gets[b] = [pltpu.async_copy(tab_s.at[sb[b].at[j]],
                                            rb[b].at[j], gsem[b])
                           for j in range(_KB)]
            for b in range(2):
                for g in gets[b]:
                    g.wait()
                puts[b] = [pltpu.async_copy(rb[b].at[j],
                                            acc.at[db[b].at[j]], ssem[b],
                                            add=True)
                           for j in range(_KB)]
            for b in range(2):
                for p in puts[b]:
                    p.wait()

                @pl.when(ch0 + b + 2 < nch)
                def _pref():
                    row = base + (ch0 + b + 2) * _KB
                    pltpu.async_copy(src2d.at[pl.ds(row, _KB)], sb[b],
                                     isem[b])
                    pltpu.async_copy(dst2d.at[pl.ds(row, _KB)], db[b],
                                     isem[b])

        plsc.subcore_barrier()
        pltpu.sync_copy(acc.at[pl.ds(s * _RPS, _RPS)],
                        out.at[c, pl.ds(s * _RPS, _RPS)])

    return _agg


_agg32 = _make_agg(32, jnp.bfloat16)
_agg16 = _make_agg(16, jnp.bfloat16)


def _tc1_body(xb, w1, partb, h1s_out, dinv_out):
    deg = jnp.sum(partb[...], axis=1, keepdims=True) + 1.0
    dinv = lax.rsqrt(deg)
    h1 = jnp.dot(xb[...], w1[...], preferred_element_type=jnp.float32)
    h1s_out[...] = (h1 * dinv).astype(h1s_out.dtype)
    dinv_out[...] = dinv


def _tc2_body(p0, p1, h1s, dinv, b1, w2, out):
    agg = (p0[...].astype(jnp.float32) + p1[...].astype(jnp.float32)
           + h1s[...].astype(jnp.float32))
    z = agg * dinv[...] + b1[...]
    z = jnp.maximum(z, 0.0)
    h2 = jnp.dot(z, w2[...], preferred_element_type=jnp.float32)
    out[...] = (h2 * dinv[...]).astype(out.dtype)


def _tc3_body(q0, q1, h2s, dinv, b2, out):
    agg = (q0[...].astype(jnp.float32) + q1[...].astype(jnp.float32)
           + h2s[...].astype(jnp.float32))
    z = agg * dinv[...] + b2[...]
    m = jnp.max(z, axis=1, keepdims=True)
    lse = jnp.log(jnp.sum(jnp.exp(z - m), axis=1, keepdims=True)) + m
    out[...] = z - lse


def _row_spec(d):
    return pl.BlockSpec((_RB, d), lambda i: (i, 0))


def _full_spec(r, d):
    return pl.BlockSpec((r, d), lambda i: (0, 0))


def kernel(x, edge_index, W1, b1, W2, b2):
    ei = edge_index.astype(jnp.int32)
    src, dst = ei[0], ei[1]
    npad = _E_PAD - _E
    src2d = jnp.concatenate([src, jnp.zeros((npad,), jnp.int32)])
    src2d = src2d.reshape(_E_PAD // _B, _B)
    dst2d = jnp.concatenate([dst, jnp.full((npad,), _N, jnp.int32)])
    dst2d = dst2d.reshape(_E_PAD // _B, _B)
    zeros32 = jnp.zeros((_N, 32), jnp.bfloat16)
    zeros16 = jnp.zeros((_N, 16), jnp.bfloat16)

    part = _deg_sc(dst2d)
    part_t = part.T

    h1s, dinv = pl.pallas_call(
        _tc1_body,
        grid=(_G,),
        in_specs=[_row_spec(128), _full_spec(128, 32), _row_spec(_NW)],
        out_specs=[_row_spec(32), _row_spec(1)],
        out_shape=[jax.ShapeDtypeStruct((_N, 32), jnp.bfloat16),
                   jax.ShapeDtypeStruct((_N, 1), jnp.float32)],
    )(x, W1, part_t)

    agg1 = _agg32(h1s, src2d, dst2d, zeros32)

    h2s = pl.pallas_call(
        _tc2_body,
        grid=(_G,),
        in_specs=[_row_spec(32), _row_spec(32), _row_spec(32), _row_spec(1),
                  _full_spec(1, 32), _full_spec(32, 16)],
        out_specs=_row_spec(16),
        out_shape=jax.ShapeDtypeStruct((_N, 16), jnp.bfloat16),
    )(agg1[0], agg1[1], h1s, dinv, b1.reshape(1, 32), W2)

    agg2 = _agg16(h2s, src2d, dst2d, zeros16)

    out = pl.pallas_call(
        _tc3_body,
        grid=(_G,),
        in_specs=[_row_spec(16), _row_spec(16), _row_spec(16), _row_spec(1),
                  _full_spec(1, 16)],
        out_specs=_row_spec(16),
        out_shape=jax.ShapeDtypeStruct((_N, 16), jnp.float32),
    )(agg2[0], agg2[1], h2s, dinv, b2.reshape(1, 16))
    return out


# speedup vs baseline: 1.0881x; 1.0881x over previous
import functools

import jax
import jax.numpy as jnp
from jax import lax
from jax.experimental import pallas as pl
from jax.experimental.pallas import tpu as pltpu
from jax.experimental.pallas import tpu_sc as plsc

_N = 10000
_E = 320000
_NC, _NS, _L = 2, 16, 16
_NW = _NC * _NS
_B = 128
_KB = 8
_TB = 80
_E_PAD = _NW * _TB * _B
_TB0_AGG, _TB1_AGG = 80, 80
_TB0_DEG, _TB1_DEG = 80, 80
_NPAD = 10016
_RPS = _N // _NS
_RB = 5000
_G = _N // _RB

_mesh = plsc.VectorSubcoreMesh(core_axis_name="c", subcore_axis_name="s",
                               num_cores=_NC, num_subcores=_NS)
_sc_params = pltpu.CompilerParams(needs_layout_passes=False,
                                  use_tc_tiling_on_sc=False,
                                  skip_device_barrier=True)


@functools.partial(
    pl.kernel,
    out_type=jax.ShapeDtypeStruct((_NW, _N), jnp.float32),
    mesh=_mesh,
    scratch_types=[
        pltpu.VMEM((_NPAD,), jnp.float32),
        pltpu.VMEM((_KB, _B), jnp.int32),
        pltpu.VMEM((_KB, _B), jnp.int32),
        pltpu.SemaphoreType.DMA,
        pltpu.SemaphoreType.DMA,
    ],
    compiler_params=_sc_params,
)
def _deg_sc(dst2d, out, hist, buf0, buf1, isem0, isem1):
    bufs, isems = [buf0, buf1], [isem0, isem1]
    c = lax.axis_index("c")
    s = lax.axis_index("s")
    wid = c * _NS + s
    base = lax.select(c == 0, s * _TB0_DEG, _NS * _TB0_DEG + s * _TB1_DEG)
    nch = lax.select(c == 0, _TB0_DEG // _KB, _TB1_DEG // _KB)
    zeros16 = jnp.zeros((_L,), jnp.float32)
    ones16 = jnp.ones((_L,), jnp.float32)

    for b in range(2):
        pltpu.async_copy(dst2d.at[pl.ds(base + b * _KB, _KB)], bufs[b],
                         isems[b])

    @pl.loop(0, _NPAD // _L)
    def _zero(i):
        hist[pl.ds(i * _L, _L)] = zeros16

    @pl.loop(0, nch // 2)
    def _chunk(i):
        ch0 = i * 2
        for b in range(2):
            pltpu.make_async_copy(dst2d.at[pl.ds(0, _KB)], bufs[b],
                                  isems[b]).wait()
            for j in range(_KB):
                for v in range(_B // _L):
                    idx = bufs[b][j, pl.ds(v * _L, _L)]
                    plsc.addupdate_scatter(hist, [idx], ones16)

            @pl.when(ch0 + b + 2 < nch)
            def _pref():
                row = base + (ch0 + b + 2) * _KB
                pltpu.async_copy(dst2d.at[pl.ds(row, _KB)], bufs[b], isems[b])

    pltpu.sync_copy(hist.at[pl.ds(0, _N)], out.at[wid])


def _make_agg(d, dtype):
    @functools.partial(
        pl.kernel,
        out_type=jax.ShapeDtypeStruct((_NC, _N, d), dtype),
        mesh=_mesh,
        scratch_types=[
            pltpu.VMEM((_KB, _B), jnp.int32),
            pltpu.VMEM((_KB, _B), jnp.int32),
            pltpu.VMEM((_KB, _B), jnp.int32),
            pltpu.VMEM((_KB, _B), jnp.int32),
            pltpu.VMEM((_KB, _B, d), dtype),
            pltpu.VMEM((_KB, _B, d), dtype),
            pltpu.VMEM_SHARED((_NPAD, d), dtype),
            pltpu.VMEM_SHARED((_N, d), dtype),
            pltpu.SemaphoreType.DMA,
            pltpu.SemaphoreType.DMA,
            pltpu.SemaphoreType.DMA,
            pltpu.SemaphoreType.DMA,
            pltpu.SemaphoreType.DMA,
            pltpu.SemaphoreType.DMA,
        ],
        compiler_params=_sc_params,
    )
    def _agg(table, src2d, dst2d, zeros, out, s0, s1, d0, d1, r0, r1, acc,
             tab_s, is0, is1, gs0, gs1, ss0, ss1):
        sb, db, rb = [s0, s1], [d0, d1], [r0, r1]
        isem, gsem, ssem = [is0, is1], [gs0, gs1], [ss0, ss1]
        c = lax.axis_index("c")
        s = lax.axis_index("s")
        base = lax.select(c == 0, s * _TB0_AGG,
                          _NS * _TB0_AGG + s * _TB1_AGG)
        nch = lax.select(c == 0, _TB0_AGG // _KB, _TB1_AGG // _KB)

        for b in range(2):
            row = base + b * _KB
            pltpu.async_copy(src2d.at[pl.ds(row, _KB)], sb[b], isem[b])
            pltpu.async_copy(dst2d.at[pl.ds(row, _KB)], db[b], isem[b])
        pltpu.sync_copy(zeros.at[pl.ds(s * _RPS, _RPS)],
                        acc.at[pl.ds(s * _RPS, _RPS)])
        pltpu.sync_copy(table.at[pl.ds(s * _RPS, _RPS)],
                        tab_s.at[pl.ds(s * _RPS, _RPS)])
        plsc.subcore_barrier()

        @pl.loop(0, nch // 2)
        def _chunk(i):
            ch0 = i * 2
            gets = [None, None]
            puts = [None, None]
            for b in range(2):
                pltpu.make_async_copy(src2d.at[pl.ds(0, _KB)], sb[b],
                                      isem[b]).wait()
                pltpu.make_async_copy(dst2d.at[pl.ds(0, _KB)], db[b],
                                      isem[b]).wait()
                gets[b] = [pltpu.async_copy(tab_s.at[sb[b].at[j]],
                                            rb[b].at[j], gsem[b])
                           for j in range(_KB)]
            for b in range(2):
                for g in gets[b]:
                    g.wait()
                puts[b] = [pltpu.async_copy(rb[b].at[j],
                                            acc.at[db[b].at[j]], ssem[b],
                                            add=True)
                           for j in range(_KB)]
            for b in range(2):
                for p in puts[b]:
                    p.wait()

                @pl.when(ch0 + b + 2 < nch)
                def _pref():
                    row = base + (ch0 + b + 2) * _KB
                    pltpu.async_copy(src2d.at[pl.ds(row, _KB)], sb[b],
                                     isem[b])
                    pltpu.async_copy(dst2d.at[pl.ds(row, _KB)], db[b],
                                     isem[b])

        plsc.subcore_barrier()
        pltpu.sync_copy(acc.at[pl.ds(s * _RPS, _RPS)],
                        out.at[c, pl.ds(s * _RPS, _RPS)])

    return _agg


_agg32 = _make_agg(32, jnp.bfloat16)
_agg16 = _make_agg(16, jnp.bfloat16)


def _tc1_body(xb, w1, partb, h1s_out, dinv_out):
    deg = jnp.sum(partb[...], axis=1, keepdims=True) + 1.0
    dinv = lax.rsqrt(deg)
    h1 = jnp.dot(xb[...], w1[...], preferred_element_type=jnp.float32)
    h1s_out[...] = (h1 * dinv).astype(h1s_out.dtype)
    dinv_out[...] = dinv


def _tc2_body(p0, p1, h1s, dinv, b1, w2, out):
    agg = (p0[...].astype(jnp.float32) + p1[...].astype(jnp.float32)
           + h1s[...].astype(jnp.float32))
    z = agg * dinv[...] + b1[...]
    z = jnp.maximum(z, 0.0)
    h2 = jnp.dot(z, w2[...], preferred_element_type=jnp.float32)
    out[...] = (h2 * dinv[...]).astype(out.dtype)


def _tc3_body(q0, q1, h2s, dinv, b2, out):
    agg = (q0[...].astype(jnp.float32) + q1[...].astype(jnp.float32)
           + h2s[...].astype(jnp.float32))
    z = agg * dinv[...] + b2[...]
    m = jnp.max(z, axis=1, keepdims=True)
    lse = jnp.log(jnp.sum(jnp.exp(z - m), axis=1, keepdims=True)) + m
    out[...] = z - lse


def _row_spec(d):
    return pl.BlockSpec((_RB, d), lambda i: (i, 0))


def _full_spec(r, d):
    return pl.BlockSpec((r, d), lambda i: (0, 0))


def kernel(x, edge_index, W1, b1, W2, b2):
    ei = edge_index.astype(jnp.int32)
    src, dst = ei[0], ei[1]
    npad = _E_PAD - _E
    src2d = jnp.concatenate([src, jnp.zeros((npad,), jnp.int32)])
    src2d = src2d.reshape(_E_PAD // _B, _B)
    dst2d = jnp.concatenate([dst, jnp.full((npad,), _N, jnp.int32)])
    dst2d = dst2d.reshape(_E_PAD // _B, _B)
    zeros32 = jnp.zeros((_N, 32), jnp.bfloat16)
    zeros16 = jnp.zeros((_N, 16), jnp.bfloat16)

    part = _deg_sc(dst2d)
    part_t = part.T

    h1s, dinv = pl.pallas_call(
        _tc1_body,
        grid=(_G,),
        in_specs=[_row_spec(128), _full_spec(128, 32), _row_spec(_NW)],
        out_specs=[_row_spec(32), _row_spec(1)],
        out_shape=[jax.ShapeDtypeStruct((_N, 32), jnp.bfloat16),
                   jax.ShapeDtypeStruct((_N, 1), jnp.float32)],
    )(x, W1, part_t)

    agg1 = _agg32(h1s, src2d, dst2d, zeros32)

    h2s = pl.pallas_call(
        _tc2_body,
        grid=(_G,),
        in_specs=[_row_spec(32), _row_spec(32), _row_spec(32), _row_spec(1),
                  _full_spec(1, 32), _full_spec(32, 16)],
        out_specs=_row_spec(16),
        out_shape=jax.ShapeDtypeStruct((_N, 16), jnp.bfloat16),
    )(agg1[0], agg1[1], h1s, dinv, b1.reshape(1, 32), W2)

    agg2 = _agg16(h2s, src2d, dst2d, zeros16)

    out = pl.pallas_call(
        _tc3_body,
        grid=(_G,),
        in_specs=[_row_spec(16), _row_spec(16), _row_spec(16), _row_spec(1),
                  _full_spec(1, 16)],
        out_specs=_row_spec(16),
        out_shape=jax.ShapeDtypeStruct((_N, 16), jnp.float32),
    )(agg2[0], agg2[1], h2s, dinv, b2.reshape(1, 16))
    return out
